# baseline (speedup 1.0000x reference)
import math
import functools

import jax
import jax.numpy as jnp
from jax.experimental import pallas as pl
from jax.experimental.pallas import tpu as pltpu

N_HEAD = 12
N_EXPERTS = 8
TOP_K = 2

BM = 256


def _ln_matmul_body(x_ref, g_ref, b_ref, w_ref, wb_ref, o_ref):
    x = x_ref[...]
    mu = jnp.mean(x, axis=1, keepdims=True)
    d = x - mu
    var = jnp.mean(d * d, axis=1, keepdims=True)
    h = d * jax.lax.rsqrt(var + 1e-5) * g_ref[...] + b_ref[...]
    acc = jax.lax.dot(h, w_ref[...], preferred_element_type=jnp.float32)
    o_ref[...] = (acc + wb_ref[...]).astype(o_ref.dtype)


def _ln_matmul(x, g, b, w, wb, out_dtype):
    N, C = x.shape
    C2 = w.shape[1]
    return pl.pallas_call(
        _ln_matmul_body,
        grid=(N // BM,),
        in_specs=[
            pl.BlockSpec((BM, C), lambda i: (i, 0)),
            pl.BlockSpec((1, C), lambda i: (0, 0)),
            pl.BlockSpec((1, C), lambda i: (0, 0)),
            pl.BlockSpec((C, C2), lambda i: (0, 0)),
            pl.BlockSpec((1, C2), lambda i: (0, 0)),
        ],
        out_specs=pl.BlockSpec((BM, C2), lambda i: (i, 0)),
        out_shape=jax.ShapeDtypeStruct((N, C2), out_dtype),
    )(x, g.reshape(1, C), b.reshape(1, C), w, wb.reshape(1, C2))


def _attn_body(q_ref, k_ref, v_ref, o_ref, *, scale, T):
    i = pl.program_id(2)
    q = q_ref[0]
    k = k_ref[0]
    v = v_ref[0]
    s = jax.lax.dot_general(q, k, (((1,), (1,)), ((), ())),
                            preferred_element_type=jnp.float32) * scale
    qidx = i * BM + jax.lax.broadcasted_iota(jnp.int32, (BM, T), 0)
    kidx = jax.lax.broadcasted_iota(jnp.int32, (BM, T), 1)
    s = jnp.where(kidx <= qidx, s, -1e30)
    m = jnp.max(s, axis=1, keepdims=True)
    p = jnp.exp(s - m)
    p = p / jnp.sum(p, axis=1, keepdims=True)
    o_ref[0] = jax.lax.dot(p, v,
                           preferred_element_type=jnp.float32).astype(o_ref.dtype)


def _attention(qkv, B, T, C):
    hd = C // N_HEAD
    N = B * T
    scale = 1.0 / math.sqrt(hd)
    nq = T // BM
    y = pl.pallas_call(
        functools.partial(_attn_body, scale=scale, T=T),
        grid=(B, N_HEAD, nq),
        in_specs=[
            pl.BlockSpec((1, BM, hd), lambda b, h, i: (h, b * nq + i, 0)),
            pl.BlockSpec((1, T, hd), lambda b, h, i: (N_HEAD + h, b, 0)),
            pl.BlockSpec((1, T, hd), lambda b, h, i: (2 * N_HEAD + h, b, 0)),
        ],
        out_specs=pl.BlockSpec((1, BM, hd), lambda b, h, i: (h, b * nq + i, 0)),
        out_shape=jax.ShapeDtypeStruct((N_HEAD, N, hd), jnp.float32),
    )(qkv, qkv, qkv)
    return y.transpose(1, 0, 2).reshape(N, C)


def _proj_res_body(y_ref, w_ref, b_ref, x_ref, o_ref):
    acc = jax.lax.dot(y_ref[...], w_ref[...],
                      preferred_element_type=jnp.float32)
    o_ref[...] = acc + b_ref[...] + x_ref[...]


def _proj_residual(y, w, b, x):
    N, C = x.shape
    return pl.pallas_call(
        _proj_res_body,
        grid=(N // BM,),
        in_specs=[
            pl.BlockSpec((BM, C), lambda i: (i, 0)),
            pl.BlockSpec((C, C), lambda i: (0, 0)),
            pl.BlockSpec((1, C), lambda i: (0, 0)),
            pl.BlockSpec((BM, C), lambda i: (i, 0)),
        ],
        out_specs=pl.BlockSpec((BM, C), lambda i: (i, 0)),
        out_shape=jax.ShapeDtypeStruct((N, C), jnp.float32),
    )(y, w, b.reshape(1, C), x)


def _router_body(x_ref, g_ref, b_ref, rw_ref, h_ref, comb_ref):
    x = x_ref[...]
    mu = jnp.mean(x, axis=1, keepdims=True)
    d = x - mu
    var = jnp.mean(d * d, axis=1, keepdims=True)
    h = d * jax.lax.rsqrt(var + 1e-5) * g_ref[...] + b_ref[...]
    h_ref[...] = h
    logits = jax.lax.dot(h, rw_ref[...],
                         preferred_element_type=jnp.float32)
    lane = jax.lax.broadcasted_iota(jnp.int32, logits.shape, 1)
    logits = jnp.where(lane < N_EXPERTS, logits, -1e30)
    v0 = jnp.max(logits, axis=1, keepdims=True)
    i0 = jnp.min(jnp.where(logits == v0, lane, 127), axis=1, keepdims=True)
    l2 = jnp.where(lane == i0, -1e30, logits)
    v1 = jnp.max(l2, axis=1, keepdims=True)
    i1 = jnp.min(jnp.where(l2 == v1, lane, 127), axis=1, keepdims=True)
    e1 = jnp.exp(v1 - v0)
    p0 = 1.0 / (1.0 + e1)
    p1 = e1 * p0
    comb_ref[...] = jnp.where(lane == i0, p0, 0.0) + jnp.where(lane == i1, p1, 0.0)


def _ln2_router(x, g, b, rw_pad):
    N, C = x.shape
    return pl.pallas_call(
        _router_body,
        grid=(N // BM,),
        in_specs=[
            pl.BlockSpec((BM, C), lambda i: (i, 0)),
            pl.BlockSpec((1, C), lambda i: (0, 0)),
            pl.BlockSpec((1, C), lambda i: (0, 0)),
            pl.BlockSpec((C, 128), lambda i: (0, 0)),
        ],
        out_specs=[
            pl.BlockSpec((BM, C), lambda i: (i, 0)),
            pl.BlockSpec((BM, 128), lambda i: (i, 0)),
        ],
        out_shape=[
            jax.ShapeDtypeStruct((N, C), jnp.float32),
            jax.ShapeDtypeStruct((N, 128), jnp.float32),
        ],
    )(x, g.reshape(1, C), b.reshape(1, C), rw_pad)


def _gelu(x):
    c = math.sqrt(2.0 / math.pi)
    return 0.5 * x * (1.0 + jnp.tanh(c * (x + 0.044715 * x * x * x)))


def _moe_body(h_ref, w1_ref, b1_ref, w2_ref, b2_ref, comb_ref, o_ref, *, nb):
    e = pl.program_id(0)
    i = pl.program_id(1)
    lane = jax.lax.broadcasted_iota(jnp.int32, comb_ref.shape, 1)
    coeff = jnp.sum(jnp.where(lane == e, comb_ref[...], 0.0), axis=1,
                    keepdims=True)
    xb = h_ref[...].astype(jnp.bfloat16)
    hid = jax.lax.dot(xb, w1_ref[0], preferred_element_type=jnp.float32)
    hid = _gelu(hid + b1_ref[0])
    out = jax.lax.dot(hid.astype(jnp.bfloat16), w2_ref[0],
                      preferred_element_type=jnp.float32)
    contrib = coeff * (out + b2_ref[0])
    rows = pl.ds(pl.multiple_of(i * BM, BM), BM)

    @pl.when(e == 0)
    def _():
        o_ref[rows, :] = contrib

    @pl.when(e > 0)
    def _():
        o_ref[rows, :] = o_ref[rows, :] + contrib


def _moe_dense(h2, w1, b1, w2, b2, comb):
    N, C = h2.shape
    E, _, H = w1.shape
    nb = N // BM
    return pl.pallas_call(
        functools.partial(_moe_body, nb=nb),
        grid=(E, nb),
        in_specs=[
            pl.BlockSpec((BM, C), lambda e, i: (i, 0)),
            pl.BlockSpec((1, C, H), lambda e, i: (e, 0, 0)),
            pl.BlockSpec((1, 1, H), lambda e, i: (e, 0, 0)),
            pl.BlockSpec((1, H, C), lambda e, i: (e, 0, 0)),
            pl.BlockSpec((1, 1, C), lambda e, i: (e, 0, 0)),
            pl.BlockSpec((BM, 128), lambda e, i: (i, 0)),
        ],
        out_specs=pl.BlockSpec((N, C), lambda e, i: (0, 0)),
        out_shape=jax.ShapeDtypeStruct((N, C), jnp.float32),
    )(h2, w1.astype(jnp.bfloat16), b1.reshape(E, 1, H),
      w2.astype(jnp.bfloat16), b2.reshape(E, 1, C), comb)


def kernel(x, ln1_g, ln1_b, c_attn_w, c_attn_b, c_proj_w, c_proj_b,
           ln2_g, ln2_b, router_w, w1, b1, w2, b2):
    B, T, C = x.shape
    N = B * T
    xf = x.reshape(N, C)
    qkv = _ln_matmul(xf, ln1_g, ln1_b, c_attn_w, c_attn_b, jnp.float32)
    hd = C // N_HEAD
    qkv_h = qkv.reshape(N, 3 * N_HEAD, hd).transpose(1, 0, 2)
    y = _attention(qkv_h, B, T, C)
    x1 = _proj_residual(y, c_proj_w, c_proj_b, xf)
    rw_pad = jnp.pad(router_w, ((0, 0), (0, 128 - N_EXPERTS)))
    h2, comb = _ln2_router(x1, ln2_g, ln2_b, rw_pad)
    y_moe = _moe_dense(h2, w1, b1, w2, b2, comb)
    return (x1 + y_moe).reshape(B, T, C)


# speedup vs baseline: 1.0860x; 1.0860x over previous
import math
import functools

import jax
import jax.numpy as jnp
from jax import lax
from jax.experimental import pallas as pl
from jax.experimental.pallas import tpu as pltpu
from jax.experimental.pallas import tpu_sc as plsc

N_HEAD = 12
N_EXPERTS = 8
TOP_K = 2

BM = 256
BM2 = 256
NP = 8192 + N_EXPERTS * BM2
NB = NP // BM2


def _ln_matmul_body(x_ref, g_ref, b_ref, w_ref, wb_ref, o_ref):
    x = x_ref[...]
    mu = jnp.mean(x, axis=1, keepdims=True)
    d = x - mu
    var = jnp.mean(d * d, axis=1, keepdims=True)
    h = d * jax.lax.rsqrt(var + 1e-5) * g_ref[...] + b_ref[...]
    acc = jax.lax.dot(h, w_ref[...], preferred_element_type=jnp.float32)
    o_ref[...] = (acc + wb_ref[...]).astype(o_ref.dtype)


def _ln_matmul(x, g, b, w, wb, out_dtype):
    N, C = x.shape
    C2 = w.shape[1]
    return pl.pallas_call(
        _ln_matmul_body,
        grid=(N // BM,),
        in_specs=[
            pl.BlockSpec((BM, C), lambda i: (i, 0)),
            pl.BlockSpec((1, C), lambda i: (0, 0)),
            pl.BlockSpec((1, C), lambda i: (0, 0)),
            pl.BlockSpec((C, C2), lambda i: (0, 0)),
            pl.BlockSpec((1, C2), lambda i: (0, 0)),
        ],
        out_specs=pl.BlockSpec((BM, C2), lambda i: (i, 0)),
        out_shape=jax.ShapeDtypeStruct((N, C2), out_dtype),
    )(x, g.reshape(1, C), b.reshape(1, C), w, wb.reshape(1, C2))


def _attn_body(q_ref, k_ref, v_ref, o_ref, *, scale, T):
    i = pl.program_id(2)
    q = q_ref[0]
    k = k_ref[0]
    v = v_ref[0]
    s = jax.lax.dot_general(q, k, (((1,), (1,)), ((), ())),
                            preferred_element_type=jnp.float32) * scale
    qidx = i * BM + jax.lax.broadcasted_iota(jnp.int32, (BM, T), 0)
    kidx = jax.lax.broadcasted_iota(jnp.int32, (BM, T), 1)
    s = jnp.where(kidx <= qidx, s, -1e30)
    m = jnp.max(s, axis=1, keepdims=True)
    p = jnp.exp(s - m)
    p = p / jnp.sum(p, axis=1, keepdims=True)
    o_ref[0] = jax.lax.dot(p, v,
                           preferred_element_type=jnp.float32).astype(o_ref.dtype)


def _attention(qkv, B, T, C):
    hd = C // N_HEAD
    N = B * T
    scale = 1.0 / math.sqrt(hd)
    nq = T // BM
    y = pl.pallas_call(
        functools.partial(_attn_body, scale=scale, T=T),
        grid=(B, N_HEAD, nq),
        in_specs=[
            pl.BlockSpec((1, BM, hd), lambda b, h, i: (h, b * nq + i, 0)),
            pl.BlockSpec((1, T, hd), lambda b, h, i: (N_HEAD + h, b, 0)),
            pl.BlockSpec((1, T, hd), lambda b, h, i: (2 * N_HEAD + h, b, 0)),
        ],
        out_specs=pl.BlockSpec((1, BM, hd), lambda b, h, i: (h, b * nq + i, 0)),
        out_shape=jax.ShapeDtypeStruct((N_HEAD, N, hd), jnp.float32),
    )(qkv, qkv, qkv)
    return y.transpose(1, 0, 2).reshape(N, C)


def _proj_res_body(y_ref, w_ref, b_ref, x_ref, o_ref):
    acc = jax.lax.dot(y_ref[...], w_ref[...],
                      preferred_element_type=jnp.float32)
    o_ref[...] = acc + b_ref[...] + x_ref[...]


def _proj_residual(y, w, b, x):
    N, C = x.shape
    return pl.pallas_call(
        _proj_res_body,
        grid=(N // BM,),
        in_specs=[
            pl.BlockSpec((BM, C), lambda i: (i, 0)),
            pl.BlockSpec((C, C), lambda i: (0, 0)),
            pl.BlockSpec((1, C), lambda i: (0, 0)),
            pl.BlockSpec((BM, C), lambda i: (i, 0)),
        ],
        out_specs=pl.BlockSpec((BM, C), lambda i: (i, 0)),
        out_shape=jax.ShapeDtypeStruct((N, C), jnp.float32),
    )(y, w, b.reshape(1, C), x)


def _router_body(x_ref, g_ref, b_ref, rw_ref, h_ref, sel_ref):
    x = x_ref[...]
    mu = jnp.mean(x, axis=1, keepdims=True)
    d = x - mu
    var = jnp.mean(d * d, axis=1, keepdims=True)
    h = d * jax.lax.rsqrt(var + 1e-5) * g_ref[...] + b_ref[...]
    h_ref[...] = h
    logits = jax.lax.dot(h, rw_ref[...], preferred_element_type=jnp.float32)
    lane = jax.lax.broadcasted_iota(jnp.int32, logits.shape, 1)
    logits = jnp.where(lane < N_EXPERTS, logits, -1e30)
    v0 = jnp.max(logits, axis=1, keepdims=True)
    i0 = jnp.min(jnp.where(logits == v0, lane, 127), axis=1, keepdims=True)
    l2 = jnp.where(lane == i0, -1e30, logits)
    v1 = jnp.max(l2, axis=1, keepdims=True)
    i1 = jnp.min(jnp.where(l2 == v1, lane, 127), axis=1, keepdims=True)
    e1 = jnp.exp(v1 - v0)
    p0 = 1.0 / (1.0 + e1)
    p1 = e1 * p0
    sel = jnp.where(lane == 0, i0.astype(jnp.float32), 0.0)
    sel = sel + jnp.where(lane == 1, i1.astype(jnp.float32), 0.0)
    sel = sel + jnp.where(lane == 2, p0, 0.0)
    sel = sel + jnp.where(lane == 3, p1, 0.0)
    sel_ref[...] = sel


def _ln2_router(x, g, b, rw_pad):
    N, C = x.shape
    return pl.pallas_call(
        _router_body,
        grid=(N // BM,),
        in_specs=[
            pl.BlockSpec((BM, C), lambda i: (i, 0)),
            pl.BlockSpec((1, C), lambda i: (0, 0)),
            pl.BlockSpec((1, C), lambda i: (0, 0)),
            pl.BlockSpec((C, 128), lambda i: (0, 0)),
        ],
        out_specs=[
            pl.BlockSpec((BM, C), lambda i: (i, 0)),
            pl.BlockSpec((BM, 128), lambda i: (i, 0)),
        ],
        out_shape=[
            jax.ShapeDtypeStruct((N, C), jnp.float32),
            jax.ShapeDtypeStruct((N, 128), jnp.float32),
        ],
    )(x, g.reshape(1, C), b.reshape(1, C), rw_pad)


def _route_body(i0_ref, i1_ref, d0_ref, d1_ref, meta_ref):
    i0 = i0_ref[...]
    i1 = i1_ref[...]
    R, L = i0.shape
    U = (jax.lax.broadcasted_iota(jnp.int32, (L, L), 0)
         < jax.lax.broadcasted_iota(jnp.int32, (L, L), 1)).astype(jnp.float32)
    Lt = (jax.lax.broadcasted_iota(jnp.int32, (R, R), 0)
          > jax.lax.broadcasted_iota(jnp.int32, (R, R), 1)).astype(jnp.float32)
    bidx = jax.lax.broadcasted_iota(jnp.int32, (1, 128), 1) * BM2
    d0 = jnp.zeros((R, L), jnp.int32)
    d1 = jnp.zeros((R, L), jnp.int32)
    blk = jnp.zeros((1, 128), jnp.int32)
    off = jnp.int32(0)
    for ee in range(N_EXPERTS):
        m0 = (i0 == ee)
        m1 = (i1 == ee)
        f0 = m0.astype(jnp.float32)
        f1 = m1.astype(jnp.float32)
        lp0 = jax.lax.dot(f0, U, preferred_element_type=jnp.float32)
        lp1 = jax.lax.dot(f1, U, preferred_element_type=jnp.float32)
        rs0 = jnp.sum(f0, axis=1, keepdims=True)
        rs1 = jnp.sum(f1, axis=1, keepdims=True)
        rp0 = jax.lax.dot(Lt, jnp.broadcast_to(rs0, (R, L)),
                          preferred_element_type=jnp.float32)
        rp1 = jax.lax.dot(Lt, jnp.broadcast_to(rs1, (R, L)),
                          preferred_element_type=jnp.float32)
        tot0 = jnp.sum(f0)
        cnt = (tot0 + jnp.sum(f1)).astype(jnp.int32)
        r0 = (rp0 + lp0).astype(jnp.int32)
        r1 = (tot0 + rp1 + lp1).astype(jnp.int32)
        d0 = jnp.where(m0, off + r0, d0)
        d1 = jnp.where(m1, off + r1, d1)
        blk = blk + jnp.where(bidx >= off, 1, 0)
        off = off + ((cnt + BM2 - 1) // BM2) * BM2
    d0_ref[...] = d0
    d1_ref[...] = d1
    meta_ref[...] = jnp.clip(blk - 1, 0, N_EXPERTS - 1)


def _route(i0, i1):
    R, L = i0.shape
    return pl.pallas_call(
        _route_body,
        grid=(1,),
        in_specs=[
            pl.BlockSpec((R, L), lambda i: (0, 0)),
            pl.BlockSpec((R, L), lambda i: (0, 0)),
        ],
        out_specs=[
            pl.BlockSpec((R, L), lambda i: (0, 0)),
            pl.BlockSpec((R, L), lambda i: (0, 0)),
            pl.BlockSpec((1, 128), lambda i: (0, 0)),
        ],
        out_shape=[
            jax.ShapeDtypeStruct((R, L), jnp.int32),
            jax.ShapeDtypeStruct((R, L), jnp.int32),
            jax.ShapeDtypeStruct((1, 128), jnp.int32),
        ],
    )(i0, i1)


def _sc_scatter(d0, d1, p0, p1):
    N = d0.shape[0]
    mesh = plsc.VectorSubcoreMesh(core_axis_name="c", subcore_axis_name="s")

    @functools.partial(
        pl.kernel, mesh=mesh,
        out_type=[
            jax.ShapeDtypeStruct((NP,), jnp.int32),
            jax.ShapeDtypeStruct((NP,), jnp.float32),
        ],
        scratch_types=[
            pltpu.VMEM((N,), jnp.int32),
            pltpu.VMEM((N,), jnp.int32),
            pltpu.VMEM((N,), jnp.float32),
            pltpu.VMEM((N,), jnp.float32),
            pltpu.VMEM((NP,), jnp.int32),
            pltpu.VMEM((NP,), jnp.float32),
        ],
        compiler_params=pltpu.CompilerParams(needs_layout_passes=False),
    )
    def k(d0_hbm, d1_hbm, p0_hbm, p1_hbm, st_hbm, sp_hbm,
          d0_v, d1_v, p0_v, p1_v, st_v, sp_v):
        wid = lax.axis_index("s") * 2 + lax.axis_index("c")

        @pl.when(wid == 0)
        def _():
            pltpu.sync_copy(d0_hbm, d0_v)
            pltpu.sync_copy(d1_hbm, d1_v)
            pltpu.sync_copy(p0_hbm, p0_v)
            pltpu.sync_copy(p1_hbm, p1_v)

            def init(i, carry):
                sl = pl.ds(i * 16, 16)
                st_v[sl] = jnp.zeros((16,), jnp.int32)
                sp_v[sl] = jnp.zeros((16,), jnp.float32)
                return carry

            lax.fori_loop(0, NP // 16, init, 0)

            def body(i, carry):
                sl = pl.ds(i * 16, 16)
                tok = lax.iota(jnp.int32, 16) + i * 16
                plsc.store_scatter(st_v, [d0_v[sl]], tok)
                plsc.store_scatter(sp_v, [d0_v[sl]], p0_v[sl])
                plsc.store_scatter(st_v, [d1_v[sl]], tok)
                plsc.store_scatter(sp_v, [d1_v[sl]], p1_v[sl])
                return carry

            lax.fori_loop(0, N // 16, body, 0)
            pltpu.sync_copy(st_v, st_hbm)
            pltpu.sync_copy(sp_v, sp_hbm)

    return k(d0, d1, p0, p1)


def _sc_gather(table, idx):
    NR = idx.shape[0]
    D = table.shape[1]
    NW = 32
    per_w = NR // NW
    CH = 64
    mesh = plsc.VectorSubcoreMesh(core_axis_name="c", subcore_axis_name="s")

    @functools.partial(
        pl.kernel, mesh=mesh,
        out_type=jax.ShapeDtypeStruct((NR, D), jnp.float32),
        scratch_types=[
            pltpu.VMEM((CH,), jnp.int32),
            pltpu.VMEM((CH, D), jnp.float32),
            pltpu.SemaphoreType.DMA,
        ],
    )
    def k(table_hbm, idx_hbm, out_hbm, idx_v, rows_v, sem):
        wid = lax.axis_index("s") * 2 + lax.axis_index("c")
        for c in range(per_w // CH):
            base = wid * per_w + c * CH
            pltpu.sync_copy(idx_hbm.at[pl.ds(base, CH)], idx_v)
            pltpu.async_copy(table_hbm.at[idx_v], rows_v, sem).wait()
            pltpu.sync_copy(rows_v, out_hbm.at[pl.ds(base, CH)])

    return k(table, idx)


def _gelu(x):
    c = math.sqrt(2.0 / math.pi)
    return 0.5 * x * (1.0 + jnp.tanh(c * (x + 0.044715 * x * x * x)))


def _ffn_body(blk_ref, xg_ref, w1_ref, b1_ref, w2_ref, b2_ref, sp_ref, o_ref):
    xb = xg_ref[...].astype(jnp.bfloat16)
    hid = jax.lax.dot(xb, w1_ref[0], preferred_element_type=jnp.float32)
    hid = _gelu(hid + b1_ref[0])
    out = jax.lax.dot(hid.astype(jnp.bfloat16), w2_ref[0],
                      preferred_element_type=jnp.float32)
    o_ref[...] = sp_ref[...] * (out + b2_ref[0])


def _ffn_grouped(xg, w1, b1, w2, b2, sp, blk):
    E, C, H = w1.shape
    grid_spec = pltpu.PrefetchScalarGridSpec(
        num_scalar_prefetch=1,
        grid=(NB,),
        in_specs=[
            pl.BlockSpec((BM2, C), lambda b, blk: (b, 0)),
            pl.BlockSpec((1, C, H), lambda b, blk: (blk[b], 0, 0)),
            pl.BlockSpec((1, 1, H), lambda b, blk: (blk[b], 0, 0)),
            pl.BlockSpec((1, H, C), lambda b, blk: (blk[b], 0, 0)),
            pl.BlockSpec((1, 1, C), lambda b, blk: (blk[b], 0, 0)),
            pl.BlockSpec((BM2, 1), lambda b, blk: (b, 0)),
        ],
        out_specs=pl.BlockSpec((BM2, C), lambda b, blk: (b, 0)),
    )
    return pl.pallas_call(
        _ffn_body,
        grid_spec=grid_spec,
        out_shape=jax.ShapeDtypeStruct((NP, C), jnp.float32),
    )(blk, xg, w1.astype(jnp.bfloat16), b1.reshape(E, 1, H),
      w2.astype(jnp.bfloat16), b2.reshape(E, 1, C), sp)


def _sc_gather2(pair, d0, d1):
    N = d0.shape[0]
    D = pair.shape[1]
    NW = 32
    per_w = N // NW
    CH = 64
    mesh = plsc.VectorSubcoreMesh(core_axis_name="c", subcore_axis_name="s")

    @functools.partial(
        pl.kernel, mesh=mesh,
        out_type=[
            jax.ShapeDtypeStruct((N, D), jnp.float32),
            jax.ShapeDtypeStruct((N, D), jnp.float32),
        ],
        scratch_types=[
            pltpu.VMEM((CH,), jnp.int32),
            pltpu.VMEM((CH, D), jnp.float32),
            pltpu.SemaphoreType.DMA,
        ],
    )
    def k(pair_hbm, d0_hbm, d1_hbm, g0_hbm, g1_hbm, idx_v, rows_v, sem):
        wid = lax.axis_index("s") * 2 + lax.axis_index("c")
        for c in range(per_w // CH):
            base = wid * per_w + c * CH
            pltpu.sync_copy(d0_hbm.at[pl.ds(base, CH)], idx_v)
            pltpu.async_copy(pair_hbm.at[idx_v], rows_v, sem).wait()
            pltpu.sync_copy(rows_v, g0_hbm.at[pl.ds(base, CH)])
            pltpu.sync_copy(d1_hbm.at[pl.ds(base, CH)], idx_v)
            pltpu.async_copy(pair_hbm.at[idx_v], rows_v, sem).wait()
            pltpu.sync_copy(rows_v, g1_hbm.at[pl.ds(base, CH)])

    return k(pair, d0, d1)


def _combine_body(x_ref, g0_ref, g1_ref, o_ref):
    o_ref[...] = x_ref[...] + g0_ref[...] + g1_ref[...]


def _combine(x1, g0, g1):
    N, C = x1.shape
    return pl.pallas_call(
        _combine_body,
        grid=(N // BM,),
        in_specs=[pl.BlockSpec((BM, C), lambda i: (i, 0))] * 3,
        out_specs=pl.BlockSpec((BM, C), lambda i: (i, 0)),
        out_shape=jax.ShapeDtypeStruct((N, C), jnp.float32),
    )(x1, g0, g1)


def kernel(x, ln1_g, ln1_b, c_attn_w, c_attn_b, c_proj_w, c_proj_b,
           ln2_g, ln2_b, router_w, w1, b1, w2, b2):
    B, T, C = x.shape
    N = B * T
    xf = x.reshape(N, C)
    qkv = _ln_matmul(xf, ln1_g, ln1_b, c_attn_w, c_attn_b, jnp.float32)
    hd = C // N_HEAD
    qkv_h = qkv.reshape(N, 3 * N_HEAD, hd).transpose(1, 0, 2)
    y = _attention(qkv_h, B, T, C)
    x1 = _proj_residual(y, c_proj_w, c_proj_b, xf)
    rw_pad = jnp.pad(router_w, ((0, 0), (0, 128 - N_EXPERTS)))
    h2, sel = _ln2_router(x1, ln2_g, ln2_b, rw_pad)
    i0 = sel[:, 0].astype(jnp.int32).reshape(N // 128, 128)
    i1 = sel[:, 1].astype(jnp.int32).reshape(N // 128, 128)
    p0 = sel[:, 2]
    p1 = sel[:, 3]
    d0, d1, meta = _route(i0, i1)
    d0f = d0.reshape(N)
    d1f = d1.reshape(N)
    st, sp = _sc_scatter(d0f, d1f, p0, p1)
    xg = _sc_gather(h2, st)
    blk = meta.reshape(128)[:NB]
    pair = _ffn_grouped(xg, w1, b1, w2, b2, sp.reshape(NP, 1), blk)
    g0, g1 = _sc_gather2(pair, d0f, d1f)
    out = _combine(x1, g0, g1)
    return out.reshape(B, T, C)
